# baseline (device time: 37084 ns/iter reference)
import jax
import jax.numpy as jnp
from jax import lax
from jax.experimental import pallas as pl
from jax.experimental.pallas import tpu as pltpu

N_DEV = 4


def kernel(x, w_mat):
    m_total, k_per = x.shape
    k_total, n = w_mat.shape
    m_per = m_total // N_DEV
    n_q = 4
    m_q = m_per // n_q

    def body(x_ref, w_hbm, out_hbm, gx_ref, w_vmem, acc_ref,
             send_sems, recv_sems, diag_sems, w_sems, out_sems):
        my = lax.axis_index("i")

        barrier_sem = pltpu.get_barrier_semaphore()
        for d in range(1, N_DEV):
            peer = lax.rem(my + d, N_DEV)
            pl.semaphore_signal(
                barrier_sem, inc=1,
                device_id=(peer,), device_id_type=pl.DeviceIdType.MESH,
            )
        pl.semaphore_wait(barrier_sem, N_DEV - 1)

        sends = []
        diag = lax.rem(my + 2, N_DEV)
        for h in range(n_q):
            rdma = pltpu.make_async_remote_copy(
                src_ref=x_ref.at[pl.ds(diag * m_per + h * m_q, m_q), :],
                dst_ref=gx_ref.at[my, pl.ds(h * m_q, m_q), :],
                send_sem=send_sems.at[2 + h],
                recv_sem=diag_sems.at[h],
                device_id=(diag,),
                device_id_type=pl.DeviceIdType.MESH,
            )
            rdma.start()
            sends.append(rdma)
        for d in (1, 3):
            dst = lax.rem(my + d, N_DEV)
            rdma = pltpu.make_async_remote_copy(
                src_ref=x_ref.at[pl.ds(dst * m_per, m_per), :],
                dst_ref=gx_ref.at[my],
                send_sem=send_sems.at[0 if d == 1 else 1],
                recv_sem=recv_sems.at[my],
                device_id=(dst,),
                device_id_type=pl.DeviceIdType.MESH,
            )
            rdma.start()
            sends.append(rdma)

        w_copies = []
        for slot, d in enumerate((0, 3, 1, 2)):
            j = lax.rem(my + d, N_DEV)
            cp = pltpu.make_async_copy(
                w_hbm.at[pl.ds(j * k_per, k_per), :],
                w_vmem.at[slot],
                w_sems.at[slot],
            )
            cp.start()
            w_copies.append(cp)

        w_copies[0].wait()
        acc_ref[:, :] = jnp.dot(
            x_ref[pl.ds(my * m_per, m_per), :], w_vmem[0],
            preferred_element_type=jnp.float32)

        for slot, d in enumerate((3, 1), start=1):
            j = lax.rem(my + d, N_DEV)
            recv = pltpu.make_async_remote_copy(
                src_ref=x_ref.at[pl.ds(0, m_per), :],
                dst_ref=gx_ref.at[j],
                send_sem=send_sems.at[0],
                recv_sem=recv_sems.at[j],
                device_id=(j,),
                device_id_type=pl.DeviceIdType.MESH,
            )
            recv.wait_recv()
            w_copies[slot].wait()
            acc_ref[:, :] += jnp.dot(
                gx_ref[j], w_vmem[slot], preferred_element_type=jnp.float32)

        w_copies[3].wait()
        out_stores = []
        for h in range(n_q):
            recv = pltpu.make_async_remote_copy(
                src_ref=x_ref.at[pl.ds(0, m_q), :],
                dst_ref=gx_ref.at[diag, pl.ds(h * m_q, m_q), :],
                send_sem=send_sems.at[0],
                recv_sem=diag_sems.at[h],
                device_id=(diag,),
                device_id_type=pl.DeviceIdType.MESH,
            )
            recv.wait_recv()
            rows = pl.ds(h * m_q, m_q)
            acc_ref[rows, :] += jnp.dot(
                gx_ref[diag, rows, :], w_vmem[3],
                preferred_element_type=jnp.float32)
            st = pltpu.make_async_copy(
                acc_ref.at[rows, :], out_hbm.at[rows, :], out_sems.at[h])
            st.start()
            out_stores.append(st)

        for st in out_stores:
            st.wait()
        for rdma in sends:
            rdma.wait_send()

    return pl.pallas_call(
        body,
        out_shape=jax.ShapeDtypeStruct((m_per, n), jnp.float32),
        in_specs=[
            pl.BlockSpec(memory_space=pltpu.VMEM),
            pl.BlockSpec(memory_space=pl.ANY),
        ],
        out_specs=pl.BlockSpec(memory_space=pl.ANY),
        scratch_shapes=[
            pltpu.VMEM((N_DEV, m_per, k_per), x.dtype),
            pltpu.VMEM((N_DEV, k_per, n), w_mat.dtype),
            pltpu.VMEM((m_per, n), jnp.float32),
            pltpu.SemaphoreType.DMA((6,)),
            pltpu.SemaphoreType.DMA((N_DEV,)),
            pltpu.SemaphoreType.DMA((4,)),
            pltpu.SemaphoreType.DMA((N_DEV,)),
            pltpu.SemaphoreType.DMA((4,)),
        ],
        compiler_params=pltpu.CompilerParams(collective_id=0),
    )(x, w_mat)


# device time: 32443 ns/iter; 1.1431x vs baseline; 1.1431x over previous
import jax
import jax.numpy as jnp
from jax import lax
from jax.experimental import pallas as pl
from jax.experimental.pallas import tpu as pltpu

N_DEV = 4


def kernel(x, w_mat):
    m_total, k_per = x.shape
    k_total, n = w_mat.shape
    m_per = m_total // N_DEV
    n_q = 4
    m_q = m_per // n_q

    def body(x_ref, w_hbm, out_hbm, gx_ref, w_vmem, acc_ref,
             send_sems, recv_sems, diag_sems, w_sems, out_sems):
        my = lax.axis_index("i")

        barrier_sem = pltpu.get_barrier_semaphore()
        for d in range(1, N_DEV):
            peer = lax.rem(my + d, N_DEV)
            pl.semaphore_signal(
                barrier_sem, inc=1,
                device_id=(peer,), device_id_type=pl.DeviceIdType.MESH,
            )
        pl.semaphore_wait(barrier_sem, N_DEV - 1)

        sends = []
        for d in (1, 3):
            dst = lax.rem(my + d, N_DEV)
            rdma = pltpu.make_async_remote_copy(
                src_ref=x_ref.at[pl.ds(dst * m_per, m_per), :],
                dst_ref=gx_ref.at[my],
                send_sem=send_sems.at[0 if d == 1 else 1],
                recv_sem=recv_sems.at[my],
                device_id=(dst,),
                device_id_type=pl.DeviceIdType.MESH,
            )
            rdma.start()
            sends.append(rdma)
        diag = lax.rem(my + 2, N_DEV)
        for h in range(n_q):
            rdma = pltpu.make_async_remote_copy(
                src_ref=x_ref.at[pl.ds(diag * m_per + h * m_q, m_q), :],
                dst_ref=gx_ref.at[my, pl.ds(h * m_q, m_q), :],
                send_sem=send_sems.at[2 + h],
                recv_sem=diag_sems.at[h],
                device_id=(diag,),
                device_id_type=pl.DeviceIdType.MESH,
            )
            rdma.start()
            sends.append(rdma)

        w_copies = []
        for slot, d in enumerate((0, 3, 1, 2)):
            j = lax.rem(my + d, N_DEV)
            cp = pltpu.make_async_copy(
                w_hbm.at[pl.ds(j * k_per, k_per), :],
                w_vmem.at[slot],
                w_sems.at[slot],
            )
            cp.start()
            w_copies.append(cp)

        w_copies[0].wait()
        acc_ref[:, :] = jnp.dot(
            x_ref[pl.ds(my * m_per, m_per), :], w_vmem[0],
            preferred_element_type=jnp.float32)

        for slot, d in enumerate((3, 1), start=1):
            j = lax.rem(my + d, N_DEV)
            recv = pltpu.make_async_remote_copy(
                src_ref=x_ref.at[pl.ds(0, m_per), :],
                dst_ref=gx_ref.at[j],
                send_sem=send_sems.at[0],
                recv_sem=recv_sems.at[j],
                device_id=(j,),
                device_id_type=pl.DeviceIdType.MESH,
            )
            recv.wait_recv()
            w_copies[slot].wait()
            acc_ref[:, :] += jnp.dot(
                gx_ref[j], w_vmem[slot], preferred_element_type=jnp.float32)

        w_copies[3].wait()
        out_stores = []
        for h in range(n_q):
            recv = pltpu.make_async_remote_copy(
                src_ref=x_ref.at[pl.ds(0, m_q), :],
                dst_ref=gx_ref.at[diag, pl.ds(h * m_q, m_q), :],
                send_sem=send_sems.at[0],
                recv_sem=diag_sems.at[h],
                device_id=(diag,),
                device_id_type=pl.DeviceIdType.MESH,
            )
            recv.wait_recv()
            rows = pl.ds(h * m_q, m_q)
            acc_ref[rows, :] += jnp.dot(
                gx_ref[diag, rows, :], w_vmem[3],
                preferred_element_type=jnp.float32)
            st = pltpu.make_async_copy(
                acc_ref.at[rows, :], out_hbm.at[rows, :], out_sems.at[h])
            st.start()
            out_stores.append(st)

        for st in out_stores:
            st.wait()
        for rdma in sends:
            rdma.wait_send()

    return pl.pallas_call(
        body,
        out_shape=jax.ShapeDtypeStruct((m_per, n), jnp.float32),
        in_specs=[
            pl.BlockSpec(memory_space=pltpu.VMEM),
            pl.BlockSpec(memory_space=pl.ANY),
        ],
        out_specs=pl.BlockSpec(memory_space=pl.ANY),
        scratch_shapes=[
            pltpu.VMEM((N_DEV, m_per, k_per), x.dtype),
            pltpu.VMEM((N_DEV, k_per, n), w_mat.dtype),
            pltpu.VMEM((m_per, n), jnp.float32),
            pltpu.SemaphoreType.DMA((6,)),
            pltpu.SemaphoreType.DMA((N_DEV,)),
            pltpu.SemaphoreType.DMA((4,)),
            pltpu.SemaphoreType.DMA((N_DEV,)),
            pltpu.SemaphoreType.DMA((4,)),
        ],
        compiler_params=pltpu.CompilerParams(collective_id=0),
    )(x, w_mat)
